# baseline (speedup 1.0000x reference)
import functools

import jax
import jax.numpy as jnp
import numpy as np
from jax import lax
from jax.experimental import pallas as pl
from jax.experimental.pallas import tpu as pltpu
from jax.experimental.pallas import tpu_sc as plsc

_NUM_CLASSES = 100
_BATCH = 4096
_NFG = _NUM_CLASSES - 1
_NPAD = 128
_MCOLS = 208
_MAXL = 7
_NW = 16
_BPW = _BATCH // _NW
_L = 16
_ACC = _L * _NPAD


def _tree_nodes():
    nodes = []

    def rec(lo, hi, depth):
        if hi - lo < 2:
            return
        mid = (lo + hi) // 2
        nodes.append((lo, mid, hi, depth))
        rec(lo, mid, depth + 1)
        rec(mid, hi, depth + 1)

    rec(0, _NFG, 1)
    return nodes


_NODES = _tree_nodes()
_N_NODES = len(_NODES)


@functools.lru_cache(maxsize=None)
def _static_tables():
    sample_nums = np.arange(100, 600, 5).astype(np.float64)
    weights = (1.0 - 0.999) / (1.0 - np.power(0.999, sample_nums))
    w_fg = weights[1:]

    wv0 = np.zeros(_NPAD, np.float32)
    wv1 = np.zeros(_NPAD, np.float32)
    dw = np.zeros(_NPAD, np.float32)
    a = np.zeros((_NUM_CLASSES, _MCOLS), np.float32)
    node_index = {}
    for n, (l, m, h, d) in enumerate(_NODES):
        v0 = np.mean(w_fg[l:m])
        v1 = np.mean(w_fg[m:h])
        s = v0 + v1
        wv0[n] = np.float32(v0 / s * 2.0)
        wv1[n] = np.float32(v1 / s * 2.0)
        dw[n] = np.float32(d / 10.0 + 1.0)
        a[1 + l:1 + m, 2 * n] = 1.0 / (m - l)
        a[1 + m:1 + h, 2 * n + 1] = 1.0 / (h - m)
        node_index[(l, h)] = (n, m)

    colc = np.zeros(_MAXL * 128, np.int32)
    wc = np.zeros(_MAXL * 128, np.float32)
    for t in range(_NFG):
        lo, hi, lvl = 0, _NFG, 0
        while hi - lo >= 2:
            n, m = node_index[(lo, hi)]
            i = lvl * 128 + t
            if t < m:
                colc[i], wc[i] = 2 * n, wv0[n]
                hi = m
            else:
                colc[i], wc[i] = 2 * n + 1, wv1[n]
                lo = m
            lvl += 1
    return a, dw, colc, wc


def _matmul_body(x_ref, a_ref, m_ref):
    m_ref[...] = jnp.dot(x_ref[...], a_ref[...],
                         preferred_element_type=jnp.float32)


def _sc_body(m_hbm, tgt_hbm, colc_hbm, wc_hbm, dw_hbm, res_hbm,
             slab_v, tgt_v, colc_v, wc_v, dw_v,
             accn_v, accd_v, accc_v, res_v, idx_v,
             shn, shd, shc):
    sid = lax.axis_index("s")
    base = sid * _BPW
    pltpu.sync_copy(m_hbm.at[pl.ds(base, _BPW)], slab_v)
    pltpu.sync_copy(tgt_hbm.at[pl.ds(base, _BPW)], tgt_v)
    pltpu.sync_copy(colc_hbm, colc_v)
    pltpu.sync_copy(wc_hbm, wc_v)
    pltpu.sync_copy(dw_hbm, dw_v)

    lane = lax.iota(jnp.int32, _L)
    zeros = jnp.zeros((_L,), jnp.float32)

    for j in range(_ACC // _L):
        accn_v[pl.ds(j * _L, _L)] = zeros
        accd_v[pl.ds(j * _L, _L)] = zeros
        accc_v[pl.ds(j * _L, _L)] = zeros
        idx_v[pl.ds(j * _L, _L)] = j * _L + lane

    @plsc.parallel_loop(0, _BPW // _L, step=1, unroll=4)
    def group_body(g):
        row = g * _L + lane
        t = plsc.load_gather(tgt_v, [row])
        fgf = jnp.where(t != 0, 1.0, 0.0)
        tf = jnp.maximum(t - 1, 0)
        for lvl in range(_MAXL):
            j = lvl * 128 + tf
            cc = plsc.load_gather(colc_v, [j])
            w = plsc.load_gather(wc_v, [j])
            co = cc ^ 1
            nd = lax.shift_right_logical(cc, 1)
            v = jnp.where(w > 0, fgf, 0.0)
            mc = plsc.load_gather(slab_v, [row, cc])
            mo = plsc.load_gather(slab_v, [row, co])
            d = mo - mc
            e = jnp.exp(-jnp.abs(d))
            z = e / (2.0 + e)
            z2 = z * z
            p = 1.0 + z2 * (
                (1.0 / 3.0) + z2 * (0.2 + z2 * ((1.0 / 7.0) + z2 * (1.0 / 9.0))))
            nll = jnp.maximum(d, 0.0) + 2.0 * z * p
            wv = w * v
            ai = nd * _L + lane
            plsc.addupdate_scatter(accn_v, [ai], wv * nll)
            plsc.addupdate_scatter(accd_v, [ai], wv)
            plsc.addupdate_scatter(accc_v, [ai], v)

    @pl.when(sid == 0)
    def _seed():
        pltpu.sync_copy(accn_v, shn)
        pltpu.sync_copy(accd_v, shd)
        pltpu.sync_copy(accc_v, shc)

    plsc.subcore_barrier()

    @pl.when(sid != 0)
    def _add():
        pltpu.sync_copy(accn_v, shn.at[idx_v], add=True)
        pltpu.sync_copy(accd_v, shd.at[idx_v], add=True)
        pltpu.sync_copy(accc_v, shc.at[idx_v], add=True)

    plsc.subcore_barrier()

    @pl.when(sid == 0)
    def _finish():
        pltpu.sync_copy(shn, accn_v)
        pltpu.sync_copy(shd, accd_v)
        pltpu.sync_copy(shc, accc_v)
        lt = jnp.zeros((_L,), jnp.float32)
        nc = jnp.zeros((_L,), jnp.float32)
        ts = jnp.zeros((_L,), jnp.float32)
        nfg = jnp.zeros((_L,), jnp.float32)
        for c in range(_NPAD // _L):
            tn = jnp.zeros((_L,), jnp.float32)
            td = jnp.zeros((_L,), jnp.float32)
            tc = jnp.zeros((_L,), jnp.float32)
            node16 = (c * _L + lane) * _L
            for l in range(_L):
                tn = tn + plsc.load_gather(accn_v, [node16 + l])
                td = td + plsc.load_gather(accd_v, [node16 + l])
                tc = tc + plsc.load_gather(accc_v, [node16 + l])
            ce = tn / jnp.where(td > 0, td, 1.0)
            ne = jnp.where(tc > 0, 1.0, 0.0)
            lt = lt + ne * ce * dw_v[pl.ds(c * _L, _L)]
            nc = nc + ne
            ts = ts + tc
            if c == 0:
                nfg = jnp.where(lane == 0, tc, 0.0)
        loss_total = jnp.full((_L,), lax.reduce_sum(lt, axes=(0,)))
        node_count = jnp.full((_L,), lax.reduce_sum(nc, axes=(0,)))
        total_samples = jnp.full((_L,), lax.reduce_sum(ts, axes=(0,)))
        n_fg = jnp.full((_L,), lax.reduce_sum(nfg, axes=(0,)))
        num_losses = n_fg * (_N_NODES / 2.0)
        res_v[...] = (loss_total / node_count) * (total_samples / num_losses)
        pltpu.sync_copy(res_v, res_hbm)


def kernel(outputs, targets):
    a, dw, colc, wc = _static_tables()
    tgt = targets.astype(jnp.int32)

    m = pl.pallas_call(
        _matmul_body,
        out_shape=jax.ShapeDtypeStruct((_BATCH, _MCOLS), jnp.float32),
    )(outputs, a)

    mesh = plsc.VectorSubcoreMesh(
        core_axis_name="c", subcore_axis_name="s",
        num_cores=1, num_subcores=_NW)
    f32 = jnp.float32
    sc = pl.kernel(
        _sc_body,
        compiler_params=pltpu.CompilerParams(needs_layout_passes=False),
        out_type=jax.ShapeDtypeStruct((_L,), f32),
        mesh=mesh,
        scratch_types=[
            pltpu.VMEM((_BPW, _MCOLS), f32),
            pltpu.VMEM((_BPW,), jnp.int32),
            pltpu.VMEM((_MAXL * 128,), jnp.int32),
            pltpu.VMEM((_MAXL * 128,), f32),
            pltpu.VMEM((_NPAD,), f32),
            pltpu.VMEM((_ACC,), f32),
            pltpu.VMEM((_ACC,), f32),
            pltpu.VMEM((_ACC,), f32),
            pltpu.VMEM((_L,), f32),
            pltpu.VMEM((_ACC,), jnp.int32),
            pltpu.VMEM_SHARED((_ACC,), f32),
            pltpu.VMEM_SHARED((_ACC,), f32),
            pltpu.VMEM_SHARED((_ACC,), f32),
        ],
    )
    res = sc(m, tgt, colc, wc, dw)
    return res[0]


# speedup vs baseline: 1.0425x; 1.0425x over previous
import functools

import jax
import jax.numpy as jnp
import numpy as np
from jax import lax
from jax.experimental import pallas as pl
from jax.experimental.pallas import tpu as pltpu
from jax.experimental.pallas import tpu_sc as plsc

_NUM_CLASSES = 100
_BATCH = 4096
_NFG = _NUM_CLASSES - 1
_NPAD = 128
_MCOLS = 256
_MAXL = 7
_NW = 16
_BPW = _BATCH // _NW
_L = 16
_ACC = _L * _NPAD


def _tree_nodes():
    nodes = []

    def rec(lo, hi, depth):
        if hi - lo < 2:
            return
        mid = (lo + hi) // 2
        nodes.append((lo, mid, hi, depth))
        rec(lo, mid, depth + 1)
        rec(mid, hi, depth + 1)

    rec(0, _NFG, 1)
    return nodes


_NODES = _tree_nodes()
_N_NODES = len(_NODES)


@functools.lru_cache(maxsize=None)
def _static_tables():
    sample_nums = np.arange(100, 600, 5).astype(np.float64)
    weights = (1.0 - 0.999) / (1.0 - np.power(0.999, sample_nums))
    w_fg = weights[1:]

    wv0 = np.zeros(_NPAD, np.float32)
    wv1 = np.zeros(_NPAD, np.float32)
    dw = np.zeros(_NPAD, np.float32)
    a = np.zeros((_NUM_CLASSES, _MCOLS), np.float32)
    node_index = {}
    for n, (l, m, h, d) in enumerate(_NODES):
        v0 = np.mean(w_fg[l:m])
        v1 = np.mean(w_fg[m:h])
        s = v0 + v1
        wv0[n] = np.float32(v0 / s * 2.0)
        wv1[n] = np.float32(v1 / s * 2.0)
        dw[n] = np.float32(d / 10.0 + 1.0)
        a[1 + l:1 + m, 2 * n] = 1.0 / (m - l)
        a[1 + m:1 + h, 2 * n + 1] = 1.0 / (h - m)
        node_index[(l, h)] = (n, m)

    colc = np.zeros(_MAXL * 128, np.int32)
    wc = np.zeros(_MAXL * 128, np.float32)
    for t in range(_NFG):
        lo, hi, lvl = 0, _NFG, 0
        while hi - lo >= 2:
            n, m = node_index[(lo, hi)]
            i = lvl * 128 + t
            if t < m:
                colc[i], wc[i] = 2 * n, wv0[n]
                hi = m
            else:
                colc[i], wc[i] = 2 * n + 1, wv1[n]
                lo = m
            lvl += 1
    return a, dw, colc, wc


def _matmul_body(x_ref, a_ref, m_ref):
    m_ref[...] = jnp.dot(x_ref[...], a_ref[...],
                         preferred_element_type=jnp.float32)


def _sc_body(m_hbm, tgt_hbm, colc_hbm, wc_hbm, dw_hbm, res_hbm,
             slab_v, tgt_v, colc_v, wc_v, dw_v,
             accn_v, accd_v, accc_v, res_v, idx_v,
             shn, shd, shc):
    sid = lax.axis_index("s")
    base = sid * _BPW
    pltpu.sync_copy(m_hbm.at[pl.ds(base, _BPW)], slab_v)
    pltpu.sync_copy(tgt_hbm.at[pl.ds(base, _BPW)], tgt_v)
    pltpu.sync_copy(colc_hbm, colc_v)
    pltpu.sync_copy(wc_hbm, wc_v)
    pltpu.sync_copy(dw_hbm, dw_v)

    lane = lax.iota(jnp.int32, _L)
    zeros = jnp.zeros((_L,), jnp.float32)

    for j in range(_ACC // _L):
        accn_v[pl.ds(j * _L, _L)] = zeros
        accd_v[pl.ds(j * _L, _L)] = zeros
        accc_v[pl.ds(j * _L, _L)] = zeros
        idx_v[pl.ds(j * _L, _L)] = j * _L + lane

    @plsc.parallel_loop(0, _BPW // _L, step=1, unroll=2)
    def group_body(g):
        row = g * _L + lane
        t = plsc.load_gather(tgt_v, [row])
        fgf = jnp.where(t != 0, 1.0, 0.0)
        tf = jnp.maximum(t - 1, 0)
        for lvl in range(_MAXL):
            j = lvl * 128 + tf
            cc = plsc.load_gather(colc_v, [j])
            w = plsc.load_gather(wc_v, [j])
            co = cc ^ 1
            nd = lax.shift_right_logical(cc, 1)
            v = jnp.where(w > 0, fgf, 0.0)
            mc = plsc.load_gather(slab_v, [row, cc])
            mo = plsc.load_gather(slab_v, [row, co])
            d = mo - mc
            e = jnp.exp(-jnp.abs(d))
            z = e / (2.0 + e)
            z2 = z * z
            p = 1.0 + z2 * (
                (1.0 / 3.0) + z2 * (0.2 + z2 * ((1.0 / 7.0) + z2 * (1.0 / 9.0))))
            nll = jnp.maximum(d, 0.0) + 2.0 * z * p
            wv = w * v
            ai = nd * _L + lane
            plsc.addupdate_scatter(accn_v, [ai], wv * nll)
            plsc.addupdate_scatter(accd_v, [ai], wv)
            plsc.addupdate_scatter(accc_v, [ai], v)

    @pl.when(sid == 0)
    def _seed():
        pltpu.sync_copy(accn_v, shn)
        pltpu.sync_copy(accd_v, shd)
        pltpu.sync_copy(accc_v, shc)

    plsc.subcore_barrier()

    @pl.when(sid != 0)
    def _add():
        pltpu.sync_copy(accn_v, shn.at[idx_v], add=True)
        pltpu.sync_copy(accd_v, shd.at[idx_v], add=True)
        pltpu.sync_copy(accc_v, shc.at[idx_v], add=True)

    plsc.subcore_barrier()

    @pl.when(sid == 0)
    def _finish():
        pltpu.sync_copy(shn, accn_v)
        pltpu.sync_copy(shd, accd_v)
        pltpu.sync_copy(shc, accc_v)
        lt = jnp.zeros((_L,), jnp.float32)
        nc = jnp.zeros((_L,), jnp.float32)
        ts = jnp.zeros((_L,), jnp.float32)
        nfg = jnp.zeros((_L,), jnp.float32)
        for c in range(_NPAD // _L):
            tn = jnp.zeros((_L,), jnp.float32)
            td = jnp.zeros((_L,), jnp.float32)
            tc = jnp.zeros((_L,), jnp.float32)
            node16 = (c * _L + lane) * _L
            for l in range(_L):
                tn = tn + plsc.load_gather(accn_v, [node16 + l])
                td = td + plsc.load_gather(accd_v, [node16 + l])
                tc = tc + plsc.load_gather(accc_v, [node16 + l])
            ce = tn / jnp.where(td > 0, td, 1.0)
            ne = jnp.where(tc > 0, 1.0, 0.0)
            lt = lt + ne * ce * dw_v[pl.ds(c * _L, _L)]
            nc = nc + ne
            ts = ts + tc
            if c == 0:
                nfg = jnp.where(lane == 0, tc, 0.0)
        loss_total = jnp.full((_L,), lax.reduce_sum(lt, axes=(0,)))
        node_count = jnp.full((_L,), lax.reduce_sum(nc, axes=(0,)))
        total_samples = jnp.full((_L,), lax.reduce_sum(ts, axes=(0,)))
        n_fg = jnp.full((_L,), lax.reduce_sum(nfg, axes=(0,)))
        num_losses = n_fg * (_N_NODES / 2.0)
        res_v[...] = (loss_total / node_count) * (total_samples / num_losses)
        pltpu.sync_copy(res_v, res_hbm)


def kernel(outputs, targets):
    a, dw, colc, wc = _static_tables()
    tgt = targets.astype(jnp.int32)

    m = pl.pallas_call(
        _matmul_body,
        out_shape=jax.ShapeDtypeStruct((_BATCH, _MCOLS), jnp.float32),
    )(outputs, a)

    mesh = plsc.VectorSubcoreMesh(
        core_axis_name="c", subcore_axis_name="s",
        num_cores=1, num_subcores=_NW)
    f32 = jnp.float32
    sc = pl.kernel(
        _sc_body,
        compiler_params=pltpu.CompilerParams(needs_layout_passes=False),
        out_type=jax.ShapeDtypeStruct((_L,), f32),
        mesh=mesh,
        scratch_types=[
            pltpu.VMEM((_BPW, _MCOLS), f32),
            pltpu.VMEM((_BPW,), jnp.int32),
            pltpu.VMEM((_MAXL * 128,), jnp.int32),
            pltpu.VMEM((_MAXL * 128,), f32),
            pltpu.VMEM((_NPAD,), f32),
            pltpu.VMEM((_ACC,), f32),
            pltpu.VMEM((_ACC,), f32),
            pltpu.VMEM((_ACC,), f32),
            pltpu.VMEM((_L,), f32),
            pltpu.VMEM((_ACC,), jnp.int32),
            pltpu.VMEM_SHARED((_ACC,), f32),
            pltpu.VMEM_SHARED((_ACC,), f32),
            pltpu.VMEM_SHARED((_ACC,), f32),
        ],
    )
    res = sc(m, tgt, colc, wc, dw)
    return res[0]


# speedup vs baseline: 1.1062x; 1.0611x over previous
import functools

import jax
import jax.numpy as jnp
import numpy as np
from jax import lax
from jax.experimental import pallas as pl
from jax.experimental.pallas import tpu as pltpu
from jax.experimental.pallas import tpu_sc as plsc

_NUM_CLASSES = 100
_BATCH = 4096
_NFG = _NUM_CLASSES - 1
_NPAD = 128
_MCOLS = 256
_MAXL = 7
_NW = 16
_BPW = _BATCH // _NW
_L = 16
_ACC = _L * _NPAD


def _tree_nodes():
    nodes = []

    def rec(lo, hi, depth):
        if hi - lo < 2:
            return
        mid = (lo + hi) // 2
        nodes.append((lo, mid, hi, depth))
        rec(lo, mid, depth + 1)
        rec(mid, hi, depth + 1)

    rec(0, _NFG, 1)
    return nodes


_NODES = _tree_nodes()
_N_NODES = len(_NODES)


@functools.lru_cache(maxsize=None)
def _static_tables():
    sample_nums = np.arange(100, 600, 5).astype(np.float64)
    weights = (1.0 - 0.999) / (1.0 - np.power(0.999, sample_nums))
    w_fg = weights[1:]

    wv0 = np.zeros(_NPAD, np.float32)
    wv1 = np.zeros(_NPAD, np.float32)
    dw = np.zeros(_NPAD, np.float32)
    a = np.zeros((_NUM_CLASSES, _MCOLS), np.float32)
    node_index = {}
    for n, (l, m, h, d) in enumerate(_NODES):
        v0 = np.mean(w_fg[l:m])
        v1 = np.mean(w_fg[m:h])
        s = v0 + v1
        wv0[n] = np.float32(v0 / s * 2.0)
        wv1[n] = np.float32(v1 / s * 2.0)
        dw[n] = np.float32(d / 10.0 + 1.0)
        a[1 + l:1 + m, 2 * n] = 1.0 / (m - l)
        a[1 + m:1 + h, 2 * n + 1] = 1.0 / (h - m)
        node_index[(l, h)] = (n, m)

    colc = np.zeros(_MAXL * 128, np.int32)
    wc = np.zeros(_MAXL * 128, np.float32)
    for t in range(_NFG):
        lo, hi, lvl = 0, _NFG, 0
        while hi - lo >= 2:
            n, m = node_index[(lo, hi)]
            i = lvl * 128 + t
            if t < m:
                colc[i], wc[i] = 2 * n, wv0[n]
                hi = m
            else:
                colc[i], wc[i] = 2 * n + 1, wv1[n]
                lo = m
            lvl += 1
    return a, dw, colc, wc


def _matmul_body(x_ref, a_ref, m_ref):
    m_ref[...] = jnp.dot(x_ref[...], a_ref[...],
                         preferred_element_type=jnp.float32)


def _sc_body(m_hbm, tgt_hbm, colc_hbm, wc_hbm, dw_hbm, res_hbm,
             slab_v, tgt_v, colc_v, wc_v, dw_v,
             accn_v, accd_v, accc_v, res_v, idx_v,
             shn, shd, shc, sem):
    sid = lax.axis_index("s")
    base = sid * _BPW
    copies = [
        pltpu.async_copy(m_hbm.at[pl.ds(base, _BPW)], slab_v, sem),
        pltpu.async_copy(tgt_hbm.at[pl.ds(base, _BPW)], tgt_v, sem),
        pltpu.async_copy(colc_hbm, colc_v, sem),
        pltpu.async_copy(wc_hbm, wc_v, sem),
        pltpu.async_copy(dw_hbm, dw_v, sem),
    ]
    for cp in copies:
        cp.wait()

    lane = lax.iota(jnp.int32, _L)
    zeros = jnp.zeros((_L,), jnp.float32)

    for j in range(_ACC // _L):
        accn_v[pl.ds(j * _L, _L)] = zeros
        accd_v[pl.ds(j * _L, _L)] = zeros
        accc_v[pl.ds(j * _L, _L)] = zeros
        idx_v[pl.ds(j * _L, _L)] = j * _L + lane

    @plsc.parallel_loop(0, _BPW // _L, step=1, unroll=2)
    def group_body(g):
        row = g * _L + lane
        t = plsc.load_gather(tgt_v, [row])
        fgf = jnp.where(t != 0, 1.0, 0.0)
        tf = jnp.maximum(t - 1, 0)
        for lvl in range(_MAXL):
            j = lvl * 128 + tf
            cc = plsc.load_gather(colc_v, [j])
            w = plsc.load_gather(wc_v, [j])
            co = cc ^ 1
            nd = lax.shift_right_logical(cc, 1)
            v = jnp.where(w > 0, fgf, 0.0)
            mc = plsc.load_gather(slab_v, [row, cc])
            mo = plsc.load_gather(slab_v, [row, co])
            d = mo - mc
            e = jnp.exp(-jnp.abs(d))
            z = e / (2.0 + e)
            z2 = z * z
            p = 1.0 + z2 * (
                (1.0 / 3.0) + z2 * (0.2 + z2 * ((1.0 / 7.0) + z2 * (1.0 / 9.0))))
            nll = jnp.maximum(d, 0.0) + 2.0 * z * p
            wv = w * v
            ai = nd * _L + lane
            plsc.addupdate_scatter(accn_v, [ai], wv * nll)
            plsc.addupdate_scatter(accd_v, [ai], wv)
            plsc.addupdate_scatter(accc_v, [ai], v)

    @pl.when(sid == 0)
    def _seed():
        pltpu.sync_copy(accn_v, shn)
        pltpu.sync_copy(accd_v, shd)
        pltpu.sync_copy(accc_v, shc)

    plsc.subcore_barrier()

    @pl.when(sid != 0)
    def _add():
        pltpu.sync_copy(accn_v, shn.at[idx_v], add=True)
        pltpu.sync_copy(accd_v, shd.at[idx_v], add=True)
        pltpu.sync_copy(accc_v, shc.at[idx_v], add=True)

    plsc.subcore_barrier()

    @pl.when(sid == 0)
    def _finish():
        pltpu.sync_copy(shn, accn_v)
        pltpu.sync_copy(shd, accd_v)
        pltpu.sync_copy(shc, accc_v)
        lt = jnp.zeros((_L,), jnp.float32)
        nc = jnp.zeros((_L,), jnp.float32)
        ts = jnp.zeros((_L,), jnp.float32)
        nfg = jnp.zeros((_L,), jnp.float32)
        for c in range(_NPAD // _L):
            tn = jnp.zeros((_L,), jnp.float32)
            td = jnp.zeros((_L,), jnp.float32)
            tc = jnp.zeros((_L,), jnp.float32)
            node16 = (c * _L + lane) * _L
            for l in range(_L):
                tn = tn + plsc.load_gather(accn_v, [node16 + l])
                td = td + plsc.load_gather(accd_v, [node16 + l])
                tc = tc + plsc.load_gather(accc_v, [node16 + l])
            ce = tn / jnp.where(td > 0, td, 1.0)
            ne = jnp.where(tc > 0, 1.0, 0.0)
            lt = lt + ne * ce * dw_v[pl.ds(c * _L, _L)]
            nc = nc + ne
            ts = ts + tc
            if c == 0:
                nfg = jnp.where(lane == 0, tc, 0.0)
        loss_total = jnp.full((_L,), lax.reduce_sum(lt, axes=(0,)))
        node_count = jnp.full((_L,), lax.reduce_sum(nc, axes=(0,)))
        total_samples = jnp.full((_L,), lax.reduce_sum(ts, axes=(0,)))
        n_fg = jnp.full((_L,), lax.reduce_sum(nfg, axes=(0,)))
        num_losses = n_fg * (_N_NODES / 2.0)
        res_v[...] = (loss_total / node_count) * (total_samples / num_losses)
        pltpu.sync_copy(res_v, res_hbm)


def kernel(outputs, targets):
    a, dw, colc, wc = _static_tables()
    tgt = targets.astype(jnp.int32)

    m = pl.pallas_call(
        _matmul_body,
        out_shape=jax.ShapeDtypeStruct((_BATCH, _MCOLS), jnp.float32),
    )(outputs, a)

    mesh = plsc.VectorSubcoreMesh(
        core_axis_name="c", subcore_axis_name="s",
        num_cores=1, num_subcores=_NW)
    f32 = jnp.float32
    sc = pl.kernel(
        _sc_body,
        compiler_params=pltpu.CompilerParams(needs_layout_passes=False),
        out_type=jax.ShapeDtypeStruct((_L,), f32),
        mesh=mesh,
        scratch_types=[
            pltpu.VMEM((_BPW, _MCOLS), f32),
            pltpu.VMEM((_BPW,), jnp.int32),
            pltpu.VMEM((_MAXL * 128,), jnp.int32),
            pltpu.VMEM((_MAXL * 128,), f32),
            pltpu.VMEM((_NPAD,), f32),
            pltpu.VMEM((_ACC,), f32),
            pltpu.VMEM((_ACC,), f32),
            pltpu.VMEM((_ACC,), f32),
            pltpu.VMEM((_L,), f32),
            pltpu.VMEM((_ACC,), jnp.int32),
            pltpu.VMEM_SHARED((_ACC,), f32),
            pltpu.VMEM_SHARED((_ACC,), f32),
            pltpu.VMEM_SHARED((_ACC,), f32),
            pltpu.SemaphoreType.DMA,
        ],
    )
    res = sc(m, tgt, colc, wc, dw)
    return res[0]
